# Initial kernel scaffold; baseline (speedup 1.0000x reference)
import jax
import jax.numpy as jnp
from jax.experimental import pallas as pl


def kernel(relation_indices, table):
    raise NotImplementedError("write your pallas kernel here")


# speedup vs baseline: 6.4823x; 6.4823x over previous
import functools

import jax
import jax.numpy as jnp
from jax import lax
from jax.experimental import pallas as pl
from jax.experimental.pallas import tpu as pltpu
from jax.experimental.pallas import tpu_sc as plsc

_VOCAB = 100000
_DIM = 32
_BATCH = 16384
_HIST = 200
_B_TOTAL = _BATCH * _HIST
_NC = 2
_NS = 16
_NW = _NC * _NS
_B_PER_W = _B_TOTAL // _NW
_CHUNK = 800
_NBUF = 4
_N_CHUNKS = _B_PER_W // _CHUNK

_mesh = plsc.VectorSubcoreMesh(core_axis_name="c", subcore_axis_name="s")


@functools.partial(
    pl.kernel,
    mesh=_mesh,
    out_type=jax.ShapeDtypeStruct((_B_TOTAL, _DIM), jnp.float32),
    scratch_types=[
        pltpu.VMEM((_NBUF, _CHUNK), jnp.int32),
        pltpu.VMEM((_NBUF, _CHUNK, _DIM), jnp.float32),
        pltpu.SemaphoreType.DMA,
        pltpu.SemaphoreType.DMA,
        pltpu.SemaphoreType.DMA,
    ],
    compiler_params=pltpu.CompilerParams(use_tc_tiling_on_sc=False),
)
def _gather_rows(idx_hbm, table_hbm, out_hbm, idx_v, rows_v, sem_g, sem_w,
                 sem_i):
    wid = lax.axis_index("s") * _NC + lax.axis_index("c")
    base = wid * _B_PER_W

    def idx_slice(i):
        return idx_hbm.at[pl.ds(base + i * _CHUNK, _CHUNK)]

    def out_slice(i):
        return out_hbm.at[pl.ds(base + i * _CHUNK, _CHUNK)]

    for b in range(_NBUF):
        pltpu.sync_copy(idx_slice(b), idx_v.at[b])
    for b in range(_NBUF - 1):
        pltpu.async_copy(table_hbm.at[idx_v.at[b]], rows_v.at[b], sem_g)

    def step(i, b):
        bj = (b - 1) % _NBUF
        pltpu.make_async_copy(table_hbm.at[idx_v.at[b]], rows_v.at[b],
                              sem_g).wait()
        pltpu.async_copy(rows_v.at[b], out_slice(i), sem_w)

        @pl.when(i + _NBUF < _N_CHUNKS)
        def _():
            pltpu.async_copy(idx_slice(i + _NBUF), idx_v.at[b], sem_i)

        @pl.when(i >= 1)
        def _():
            pltpu.make_async_copy(rows_v.at[bj], out_slice(i - 1),
                                  sem_w).wait()

        @pl.when(jnp.logical_and(i >= 1, i + _NBUF - 1 < _N_CHUNKS))
        def _():
            pltpu.make_async_copy(idx_slice(i + _NBUF - 1), idx_v.at[bj],
                                  sem_i).wait()

        @pl.when(i + _NBUF - 1 < _N_CHUNKS)
        def _():
            pltpu.async_copy(table_hbm.at[idx_v.at[bj]], rows_v.at[bj],
                             sem_g)

    def outer(o, carry):
        for b in range(_NBUF):
            step(o * _NBUF + b, b)
        return carry

    lax.fori_loop(0, _N_CHUNKS // _NBUF, outer, 0)

    last_b = (_N_CHUNKS - 1) % _NBUF
    pltpu.make_async_copy(rows_v.at[last_b], out_slice(_N_CHUNKS - 1),
                          sem_w).wait()


def kernel(relation_indices, table):
    flat = relation_indices.reshape(_B_TOTAL)
    out = _gather_rows(flat, table)
    return out.reshape(_BATCH, _HIST, _DIM)


# speedup vs baseline: 6.5023x; 1.0031x over previous
import functools

import jax
import jax.numpy as jnp
from jax import lax
from jax.experimental import pallas as pl
from jax.experimental.pallas import tpu as pltpu
from jax.experimental.pallas import tpu_sc as plsc

_VOCAB = 100000
_DIM = 32
_BATCH = 16384
_HIST = 200
_NC = 2
_NS = 16
_NW = _NC * _NS
_ROWS_PER_W = _BATCH // _NW
_CR = 4
_NBUF = 4
_N_CHUNKS = _ROWS_PER_W // _CR

_mesh = plsc.VectorSubcoreMesh(core_axis_name="c", subcore_axis_name="s")


@functools.partial(
    pl.kernel,
    mesh=_mesh,
    out_type=jax.ShapeDtypeStruct((_BATCH, _HIST, _DIM), jnp.float32),
    scratch_types=[
        pltpu.VMEM((_NBUF, _CR, _HIST), jnp.int32),
        pltpu.VMEM((_NBUF, _CR, _HIST, _DIM), jnp.float32),
        pltpu.SemaphoreType.DMA,
        pltpu.SemaphoreType.DMA,
        pltpu.SemaphoreType.DMA,
    ],
    compiler_params=pltpu.CompilerParams(use_tc_tiling_on_sc=False),
)
def _gather_rows(idx_hbm, table_hbm, out_hbm, idx_v, rows_v, sem_g, sem_w,
                 sem_i):
    wid = lax.axis_index("s") * _NC + lax.axis_index("c")
    base = wid * _ROWS_PER_W

    def idx_slice(i):
        return idx_hbm.at[pl.ds(base + i * _CR, _CR)]

    def out_slice(i):
        return out_hbm.at[pl.ds(base + i * _CR, _CR)]

    def start_gather(b):
        for r in range(_CR):
            pltpu.async_copy(table_hbm.at[idx_v.at[b].at[r]],
                             rows_v.at[b].at[r], sem_g)

    def wait_gather(b):
        for r in range(_CR):
            pltpu.make_async_copy(table_hbm.at[idx_v.at[b].at[r]],
                                  rows_v.at[b].at[r], sem_g).wait()

    for b in range(_NBUF):
        pltpu.sync_copy(idx_slice(b), idx_v.at[b])
    for b in range(_NBUF - 1):
        start_gather(b)

    def step(i, b):
        bj = (b - 1) % _NBUF
        wait_gather(b)
        pltpu.async_copy(rows_v.at[b], out_slice(i), sem_w)

        @pl.when(i + _NBUF < _N_CHUNKS)
        def _():
            pltpu.async_copy(idx_slice(i + _NBUF), idx_v.at[b], sem_i)

        @pl.when(i >= 1)
        def _():
            pltpu.make_async_copy(rows_v.at[bj], out_slice(i - 1),
                                  sem_w).wait()

        @pl.when(jnp.logical_and(i >= 1, i + _NBUF - 1 < _N_CHUNKS))
        def _():
            pltpu.make_async_copy(idx_slice(i + _NBUF - 1), idx_v.at[bj],
                                  sem_i).wait()

        @pl.when(i + _NBUF - 1 < _N_CHUNKS)
        def _():
            start_gather(bj)

    def outer(o, carry):
        for b in range(_NBUF):
            step(o * _NBUF + b, b)
        return carry

    lax.fori_loop(0, _N_CHUNKS // _NBUF, outer, 0)

    last_b = (_N_CHUNKS - 1) % _NBUF
    pltpu.make_async_copy(rows_v.at[last_b], out_slice(_N_CHUNKS - 1),
                          sem_w).wait()


def kernel(relation_indices, table):
    return _gather_rows(relation_indices, table)


# speedup vs baseline: 16.2038x; 2.4920x over previous
import functools

import jax
import jax.numpy as jnp
from jax import lax
from jax.experimental import pallas as pl
from jax.experimental.pallas import tpu as pltpu
from jax.experimental.pallas import tpu_sc as plsc

_VOCAB = 100000
_DIM = 32
_BATCH = 16384
_HIST = 200
_NC = 2
_NS = 16
_NW = _NC * _NS
_BC = 4096
_NB = _BATCH // _BC
_N_CH = _HIST * _NB

_mesh = plsc.VectorSubcoreMesh(core_axis_name="c", subcore_axis_name="s")


@functools.partial(
    pl.kernel,
    mesh=_mesh,
    out_type=jax.ShapeDtypeStruct((_HIST, _DIM, _BATCH), jnp.float32),
    scratch_types=[
        pltpu.VMEM((_VOCAB,), jnp.float32),
        pltpu.VMEM((2, _BC), jnp.int32),
        pltpu.VMEM((2, _BC), jnp.float32),
        pltpu.SemaphoreType.DMA,
        pltpu.SemaphoreType.DMA,
        pltpu.SemaphoreType.DMA,
    ],
    compiler_params=pltpu.CompilerParams(use_tc_tiling_on_sc=True,
                                         needs_layout_passes=False),
)
def _gather_cols(idx_hbm, tab_hbm, out_hbm, trow, idxb, outb, sem_t, sem_i,
                 sem_w):
    d = lax.axis_index("s") * _NC + lax.axis_index("c")

    def hb(i):
        return i // _NB, (i % _NB) * _BC

    def idx_slice(i):
        h, b0 = hb(i)
        return idx_hbm.at[h, pl.ds(b0, _BC)]

    def out_slice(i):
        h, b0 = hb(i)
        return out_hbm.at[h, d, pl.ds(b0, _BC)]

    pltpu.async_copy(tab_hbm.at[d], trow, sem_t)
    pltpu.async_copy(idx_slice(0), idxb.at[0], sem_i)
    pltpu.make_async_copy(tab_hbm.at[d], trow, sem_t).wait()

    def step(i, slot):
        pltpu.make_async_copy(idx_slice(i), idxb.at[slot], sem_i).wait()

        @pl.when(i + 1 < _N_CH)
        def _():
            pltpu.async_copy(idx_slice(i + 1), idxb.at[1 - slot], sem_i)

        @pl.when(i >= 2)
        def _():
            pltpu.make_async_copy(outb.at[slot], out_slice(i - 2),
                                  sem_w).wait()

        @plsc.parallel_loop(0, _BC, step=16, unroll=8)
        def _(k):
            iv = idxb[slot, pl.ds(k, 16)]
            outb[slot, pl.ds(k, 16)] = plsc.load_gather(trow, [iv])

        pltpu.async_copy(outb.at[slot], out_slice(i), sem_w)

    def outer(o, carry):
        step(2 * o, 0)
        step(2 * o + 1, 1)
        return carry

    lax.fori_loop(0, _N_CH // 2, outer, 0)

    pltpu.make_async_copy(outb.at[0], out_slice(_N_CH - 2), sem_w).wait()
    pltpu.make_async_copy(outb.at[1], out_slice(_N_CH - 1), sem_w).wait()


def kernel(relation_indices, table):
    out_t = _gather_cols(relation_indices.T, table.T)
    return jnp.transpose(out_t, (2, 0, 1))


# speedup vs baseline: 16.3986x; 1.0120x over previous
import functools

import jax
import jax.numpy as jnp
from jax import lax
from jax.experimental import pallas as pl
from jax.experimental.pallas import tpu as pltpu
from jax.experimental.pallas import tpu_sc as plsc

_VOCAB = 100000
_DIM = 32
_BATCH = 16384
_HIST = 200
_NC = 2
_NS = 16
_NW = _NC * _NS
_HB = 8
_BB = 512
_NSEG = _HB * (_BB // 128)
_NG = _HIST // _HB
_NJ = _BATCH // _BB
_N_CH = _NG * _NJ
_SEGS = _HIST * _DIM * _BATCH // 128

_mesh = plsc.VectorSubcoreMesh(core_axis_name="c", subcore_axis_name="s")


@functools.partial(
    pl.kernel,
    mesh=_mesh,
    out_type=jax.ShapeDtypeStruct((_SEGS, 128), jnp.float32),
    scratch_types=[
        pltpu.VMEM((_VOCAB,), jnp.float32),
        pltpu.VMEM((2, _HB, _BB), jnp.int32),
        pltpu.VMEM((2, _NSEG, 128), jnp.float32),
        pltpu.VMEM((2, _NSEG), jnp.int32),
        pltpu.SemaphoreType.DMA,
        pltpu.SemaphoreType.DMA,
        pltpu.SemaphoreType.DMA,
    ],
    compiler_params=pltpu.CompilerParams(use_tc_tiling_on_sc=True,
                                         needs_layout_passes=False),
)
def _gather_cols(idx_hbm, tab_hbm, out_hbm, trow, idxb, outb, segv, sem_t,
                 sem_i, sem_w):
    d = lax.axis_index("s") * _NC + lax.axis_index("c")
    seg_d = (d // 8) * 1024 + d % 8

    def idx_slice(i):
        g, j = i // _NJ, i % _NJ
        return idx_hbm.at[pl.ds(g * _HB, _HB), pl.ds(j * _BB, _BB)]

    def seg_base(i):
        g, j = i // _NJ, i % _NJ
        return g * _HB * 4096 + j * (_BB // 128) * 8 + seg_d

    pltpu.async_copy(tab_hbm.at[d], trow, sem_t)
    pltpu.async_copy(idx_slice(0), idxb.at[0], sem_i)
    pltpu.make_async_copy(tab_hbm.at[d], trow, sem_t).wait()

    def step(i, slot):
        pltpu.make_async_copy(idx_slice(i), idxb.at[slot], sem_i).wait()

        @pl.when(i + 1 < _N_CH)
        def _():
            pltpu.async_copy(idx_slice(i + 1), idxb.at[1 - slot], sem_i)

        @pl.when(i >= 2)
        def _():
            pltpu.make_async_copy(outb.at[slot],
                                  out_hbm.at[segv.at[slot]], sem_w).wait()

        base = seg_base(i)
        for half in range(2):
            m = lax.iota(jnp.int32, 16) + 16 * half
            segv[slot, pl.ds(16 * half, 16)] = (
                base + (m // 4) * 4096 + (m % 4) * 8)

        @plsc.parallel_loop(0, _HB * _BB, step=16, unroll=8)
        def _(k):
            iv = idxb[slot, k // _BB, pl.ds(lax.rem(k, _BB), 16)]
            outb[slot, k // 128, pl.ds(lax.rem(k, 128), 16)] = (
                plsc.load_gather(trow, [iv]))

        pltpu.async_copy(outb.at[slot], out_hbm.at[segv.at[slot]], sem_w)

    def outer(o, carry):
        step(2 * o, 0)
        step(2 * o + 1, 1)
        return carry

    lax.fori_loop(0, _N_CH // 2, outer, 0)

    pltpu.make_async_copy(outb.at[0], out_hbm.at[segv.at[0]], sem_w).wait()
    pltpu.make_async_copy(outb.at[1], out_hbm.at[segv.at[1]], sem_w).wait()


def kernel(relation_indices, table):
    out4 = _gather_cols(relation_indices.T, table.T)
    out5 = out4.reshape(_HIST, _DIM // 8, _BATCH // 128, 8, 128)
    return out5.transpose(2, 4, 0, 1, 3).reshape(_BATCH, _HIST, _DIM)


# speedup vs baseline: 23.5934x; 1.4387x over previous
import functools

import jax
import jax.numpy as jnp
from jax import lax
from jax.experimental import pallas as pl
from jax.experimental.pallas import tpu as pltpu
from jax.experimental.pallas import tpu_sc as plsc

_VOCAB = 100000
_DIM = 32
_BATCH = 16384
_HIST = 200
_NC = 2
_NS = 16
_NW = _NC * _NS
_HB = 8
_BB = 256
_NSEG = _HB * (_BB // 128)
_NG = _HIST // _HB
_NJ = _BATCH // _BB
_N_CH = _NG * _NJ
_NBUF = 4
_SEGS = _HIST * _DIM * _BATCH // 128

_mesh = plsc.VectorSubcoreMesh(core_axis_name="c", subcore_axis_name="s")


@functools.partial(
    pl.kernel,
    mesh=_mesh,
    out_type=jax.ShapeDtypeStruct((_SEGS, 128), jnp.float32),
    scratch_types=[
        pltpu.VMEM((_VOCAB,), jnp.float32),
        pltpu.VMEM((_NBUF, _HB, _BB), jnp.int32),
        pltpu.VMEM((_NBUF, _NSEG, 128), jnp.float32),
        pltpu.VMEM((_NBUF, _NSEG), jnp.int32),
        pltpu.SemaphoreType.DMA,
        pltpu.SemaphoreType.DMA,
        pltpu.SemaphoreType.DMA,
    ],
    compiler_params=pltpu.CompilerParams(use_tc_tiling_on_sc=True,
                                         needs_layout_passes=False),
)
def _gather_cols(idx_hbm, tab_hbm, out_hbm, trow, idxb, outb, segv, sem_t,
                 sem_i, sem_w):
    d = lax.axis_index("s") * _NC + lax.axis_index("c")
    seg_d = (d // 8) * 1024 + d % 8

    def idx_slice(i):
        g, j = i // _NJ, i % _NJ
        return idx_hbm.at[pl.ds(g * _HB, _HB), pl.ds(j * _BB, _BB)]

    def seg_base(i):
        g, j = i // _NJ, i % _NJ
        return g * _HB * 4096 + j * (_BB // 128) * 8 + seg_d

    pltpu.async_copy(tab_hbm.at[d], trow, sem_t)
    for b in range(_NBUF - 1):
        pltpu.async_copy(idx_slice(b), idxb.at[b], sem_i)
    pltpu.make_async_copy(tab_hbm.at[d], trow, sem_t).wait()

    def step(i, slot):
        pltpu.make_async_copy(idx_slice(i), idxb.at[slot], sem_i).wait()

        @pl.when(i + _NBUF - 1 < _N_CH)
        def _():
            pltpu.async_copy(idx_slice(i + _NBUF - 1),
                             idxb.at[(slot + _NBUF - 1) % _NBUF], sem_i)

        @pl.when(i >= _NBUF)
        def _():
            pltpu.make_async_copy(outb.at[slot],
                                  out_hbm.at[segv.at[slot]], sem_w).wait()

        m = lax.iota(jnp.int32, 16)
        segv[slot, pl.ds(0, 16)] = (
            seg_base(i) + (m // (_BB // 128)) * 4096
            + lax.rem(m, _BB // 128) * 8)

        @plsc.parallel_loop(0, _HB * _BB, step=16, unroll=8)
        def _(k):
            iv = idxb[slot, k // _BB, pl.ds(lax.rem(k, _BB), 16)]
            outb[slot, k // 128, pl.ds(lax.rem(k, 128), 16)] = (
                plsc.load_gather(trow, [iv]))

        pltpu.async_copy(outb.at[slot], out_hbm.at[segv.at[slot]], sem_w)

    def outer(o, carry):
        for b in range(_NBUF):
            step(_NBUF * o + b, b)
        return carry

    lax.fori_loop(0, _N_CH // _NBUF, outer, 0)

    for b in range(_NBUF):
        pltpu.make_async_copy(outb.at[b], out_hbm.at[segv.at[b]],
                              sem_w).wait()


def kernel(relation_indices, table):
    out4 = _gather_cols(relation_indices.T, table.T)
    out5 = out4.reshape(_HIST, _DIM // 8, _BATCH // 128, 8, 128)
    return out5.transpose(2, 4, 0, 1, 3).reshape(_BATCH, _HIST, _DIM)
